# baseline (speedup 1.0000x reference)
import functools

import jax
import jax.numpy as jnp
from jax.experimental import pallas as pl
from jax.experimental.pallas import tpu as pltpu

TBP = 128
TB = 128


def _proj_kernel(f_ref, g_ref, w_ref, bias_ref, ff_ref, gg_ref, *, H1, B):
    ti = pl.program_id(0)
    wf = w_ref[:H1, :]
    wg = w_ref[H1:, :]
    for b in range(B):
        ff_ref[b] = (
            jnp.dot(f_ref[:, b, :], wf, preferred_element_type=jnp.float32)
            + bias_ref[0]
        )

    @pl.when(ti == 0)
    def _():
        for b in range(B):
            gg_ref[b] = jnp.dot(g_ref[b], wg, preferred_element_type=jnp.float32)


def _add_kernel(lens_ref, ff_ref, gg_ref, out_ref, *, U):
    bi = pl.program_id(0)
    ti = pl.program_id(1)
    f_len = lens_ref[0, bi]
    g_len = lens_ref[1, bi]

    ff = ff_ref[0]
    gg = gg_ref[0]
    V = ff.shape[1]

    t_ids = ti * TB + jax.lax.broadcasted_iota(jnp.int32, (TB, V), 0)
    u_ids = jax.lax.broadcasted_iota(jnp.int32, (U, V), 0)
    tmask = (t_ids < f_len).astype(jnp.float32)
    umask = (u_ids < g_len).astype(jnp.float32)

    summed = ff[:, None, :] + gg[None, :, :]
    out_ref[0] = summed * tmask[:, None, :] * umask[None, :, :]


def kernel(f, f_lens, g, g_lens, W, b):
    T, B, H1 = f.shape
    _, U, H2 = g.shape
    V = W.shape[1]

    lens = jnp.stack([f_lens, g_lens]).astype(jnp.int32)
    bias2d = b.reshape(1, V)

    ff, gg = pl.pallas_call(
        functools.partial(_proj_kernel, H1=H1, B=B),
        grid=(T // TBP,),
        in_specs=[
            pl.BlockSpec((TBP, B, H1), lambda ti: (ti, 0, 0)),
            pl.BlockSpec((B, U, H2), lambda ti: (0, 0, 0)),
            pl.BlockSpec((H1 + H2, V), lambda ti: (0, 0)),
            pl.BlockSpec((1, V), lambda ti: (0, 0)),
        ],
        out_specs=[
            pl.BlockSpec((B, TBP, V), lambda ti: (0, ti, 0)),
            pl.BlockSpec((B, U, V), lambda ti: (0, 0, 0)),
        ],
        out_shape=[
            jax.ShapeDtypeStruct((B, T, V), jnp.float32),
            jax.ShapeDtypeStruct((B, U, V), jnp.float32),
        ],
    )(f, g, W, bias2d)

    out = pl.pallas_call(
        functools.partial(_add_kernel, U=U),
        grid_spec=pltpu.PrefetchScalarGridSpec(
            num_scalar_prefetch=1,
            grid=(B, T // TB),
            in_specs=[
                pl.BlockSpec((1, TB, V), lambda bi, ti, lens: (bi, ti, 0)),
                pl.BlockSpec((1, U, V), lambda bi, ti, lens: (bi, 0, 0)),
            ],
            out_specs=pl.BlockSpec((1, TB, U, V), lambda bi, ti, lens: (bi, ti, 0, 0)),
        ),
        out_shape=jax.ShapeDtypeStruct((B, T, U, V), jnp.float32),
        compiler_params=pltpu.CompilerParams(
            dimension_semantics=("parallel", "parallel"),
        ),
    )(lens, ff, gg)
    return (out, f_lens)


# speedup vs baseline: 1.0625x; 1.0625x over previous
import functools

import jax
import jax.numpy as jnp
from jax.experimental import pallas as pl
from jax.experimental.pallas import tpu as pltpu

TBP = 128
TB = 256


def _proj_kernel(f_ref, g_ref, w_ref, bias_ref, ff_ref, gg_ref, *, H1, B):
    ti = pl.program_id(0)
    wf = w_ref[:H1, :]
    wg = w_ref[H1:, :]
    for b in range(B):
        ff_ref[b] = (
            jnp.dot(f_ref[:, b, :], wf, preferred_element_type=jnp.float32)
            + bias_ref[0]
        )

    @pl.when(ti == 0)
    def _():
        for b in range(B):
            gg_ref[b] = jnp.dot(g_ref[b], wg, preferred_element_type=jnp.float32)


def _add_kernel(lens_ref, ff_ref, gg_ref, out_ref, *, U):
    bi = pl.program_id(0)
    ti = pl.program_id(1)
    f_len = lens_ref[0, bi]
    g_len = lens_ref[1, bi]

    ff = ff_ref[0]
    gg = gg_ref[0]
    V = ff.shape[1]

    t_ids = ti * TB + jax.lax.broadcasted_iota(jnp.int32, (TB, V), 0)
    u_ids = jax.lax.broadcasted_iota(jnp.int32, (U, V), 0)
    tmask = (t_ids < f_len).astype(jnp.float32)
    umask = (u_ids < g_len).astype(jnp.float32)

    summed = ff[:, None, :] + gg[None, :, :]
    out_ref[0] = summed * tmask[:, None, :] * umask[None, :, :]


def kernel(f, f_lens, g, g_lens, W, b):
    T, B, H1 = f.shape
    _, U, H2 = g.shape
    V = W.shape[1]

    lens = jnp.stack([f_lens, g_lens]).astype(jnp.int32)
    bias2d = b.reshape(1, V)

    ff, gg = pl.pallas_call(
        functools.partial(_proj_kernel, H1=H1, B=B),
        grid=(T // TBP,),
        in_specs=[
            pl.BlockSpec((TBP, B, H1), lambda ti: (ti, 0, 0)),
            pl.BlockSpec((B, U, H2), lambda ti: (0, 0, 0)),
            pl.BlockSpec((H1 + H2, V), lambda ti: (0, 0)),
            pl.BlockSpec((1, V), lambda ti: (0, 0)),
        ],
        out_specs=[
            pl.BlockSpec((B, TBP, V), lambda ti: (0, ti, 0)),
            pl.BlockSpec((B, U, V), lambda ti: (0, 0, 0)),
        ],
        out_shape=[
            jax.ShapeDtypeStruct((B, T, V), jnp.float32),
            jax.ShapeDtypeStruct((B, U, V), jnp.float32),
        ],
    )(f, g, W, bias2d)

    out = pl.pallas_call(
        functools.partial(_add_kernel, U=U),
        grid_spec=pltpu.PrefetchScalarGridSpec(
            num_scalar_prefetch=1,
            grid=(B, T // TB),
            in_specs=[
                pl.BlockSpec((1, TB, V), lambda bi, ti, lens: (bi, ti, 0)),
                pl.BlockSpec((1, U, V), lambda bi, ti, lens: (bi, 0, 0)),
            ],
            out_specs=pl.BlockSpec((1, TB, U, V), lambda bi, ti, lens: (bi, ti, 0, 0)),
        ),
        out_shape=jax.ShapeDtypeStruct((B, T, U, V), jnp.float32),
        compiler_params=pltpu.CompilerParams(
            dimension_semantics=("parallel", "parallel"),
        ),
    )(lens, ff, gg)
    return (out, f_lens)
